# baseline (speedup 1.0000x reference)
import functools

import jax
import jax.numpy as jnp
from jax import lax
from jax.experimental import pallas as pl
from jax.experimental.pallas import tpu as pltpu
from jax.experimental.pallas import tpu_sc as plsc

_BLOCK = 128


@functools.lru_cache(maxsize=None)
def _build(n_nodes: int, n_vocab: int, dim: int):
    info = plsc.get_sparse_core_info()
    nc, ns = info.num_cores, info.num_subcores
    nw = nc * ns

    nblk = n_nodes // _BLOCK
    tail = n_nodes - nblk * _BLOCK
    assert tail % 8 == 0 and (nblk * _BLOCK) % 8 == 0

    mesh = plsc.VectorSubcoreMesh(core_axis_name="c", subcore_axis_name="s")

    @functools.partial(
        pl.kernel,
        out_type=jax.ShapeDtypeStruct((n_nodes, dim), jnp.float32),
        mesh=mesh,
        scratch_types=[
            pltpu.VMEM((_BLOCK,), jnp.int32),
            pltpu.VMEM((_BLOCK, dim), jnp.float32),
            pltpu.SemaphoreType.DMA,
        ],
    )
    def emb_kernel(x_hbm, w_hbm, out_hbm, idx_v, rows_v, sem):
        wid = lax.axis_index("s") * nc + lax.axis_index("c")

        def do_block(base, n_rows):
            pltpu.sync_copy(x_hbm.at[pl.ds(base, n_rows)],
                            idx_v.at[pl.ds(0, n_rows)])
            pltpu.async_copy(w_hbm.at[idx_v.at[pl.ds(0, n_rows)]],
                             rows_v.at[pl.ds(0, n_rows)], sem).wait()
            pltpu.sync_copy(rows_v.at[pl.ds(0, n_rows)],
                            out_hbm.at[pl.ds(base, n_rows)])

        n_mine = (nblk - 1 - wid) // nw + 1

        @pl.when(wid < nblk)
        def _():
            def step(t, carry):
                do_block((wid + t * nw) * _BLOCK, _BLOCK)
                return carry
            lax.fori_loop(0, n_mine, step, 0)

        if tail:
            @pl.when(wid == nw - 1)
            def _():
                do_block(nblk * _BLOCK, tail)

    return emb_kernel


def kernel(x, weight):
    n_nodes = x.shape[0]
    n_vocab, dim = weight.shape
    emb = _build(n_nodes, n_vocab, dim)
    return emb(x.astype(jnp.int32), weight)


# speedup vs baseline: 4.2941x; 4.2941x over previous
import functools

import jax
import jax.numpy as jnp
from jax import lax
from jax.experimental import pallas as pl
from jax.experimental.pallas import tpu as pltpu
from jax.experimental.pallas import tpu_sc as plsc

_BLOCK = 128


@functools.lru_cache(maxsize=None)
def _build(n_nodes: int, n_vocab: int, dim: int):
    info = plsc.get_sparse_core_info()
    nc, ns = info.num_cores, info.num_subcores
    nw = nc * ns

    nblk = n_nodes // _BLOCK
    tail = n_nodes - nblk * _BLOCK
    assert tail % 8 == 0 and (nblk * _BLOCK) % 8 == 0

    mesh = plsc.VectorSubcoreMesh(core_axis_name="c", subcore_axis_name="s")

    @functools.partial(
        pl.kernel,
        out_type=jax.ShapeDtypeStruct((n_nodes, dim), jnp.float32),
        mesh=mesh,
        scratch_types=[
            pltpu.VMEM_SHARED((n_vocab, dim), jnp.float32),
            pltpu.VMEM((_BLOCK,), jnp.int32),
            pltpu.VMEM((_BLOCK, dim), jnp.float32),
            pltpu.SemaphoreType.DMA,
        ],
    )
    def emb_kernel(x_hbm, w_hbm, out_hbm, table_s, idx_v, rows_v, sem):
        sid = lax.axis_index("s")
        wid = sid * nc + lax.axis_index("c")

        @pl.when(sid == 0)
        def _():
            pltpu.sync_copy(w_hbm, table_s)

        plsc.subcore_barrier()

        def do_block(base, n_rows):
            pltpu.sync_copy(x_hbm.at[pl.ds(base, n_rows)],
                            idx_v.at[pl.ds(0, n_rows)])
            pltpu.async_copy(table_s.at[idx_v.at[pl.ds(0, n_rows)]],
                             rows_v.at[pl.ds(0, n_rows)], sem).wait()
            pltpu.sync_copy(rows_v.at[pl.ds(0, n_rows)],
                            out_hbm.at[pl.ds(base, n_rows)])

        n_mine = (nblk - 1 - wid) // nw + 1

        @pl.when(wid < nblk)
        def _():
            def step(t, carry):
                do_block((wid + t * nw) * _BLOCK, _BLOCK)
                return carry
            lax.fori_loop(0, n_mine, step, 0)

        if tail:
            @pl.when(wid == nw - 1)
            def _():
                do_block(nblk * _BLOCK, tail)

    return emb_kernel


def kernel(x, weight):
    n_nodes = x.shape[0]
    n_vocab, dim = weight.shape
    emb = _build(n_nodes, n_vocab, dim)
    return emb(x.astype(jnp.int32), weight)


# speedup vs baseline: 5.4458x; 1.2682x over previous
import functools

import jax
import jax.numpy as jnp
from jax import lax
from jax.experimental import pallas as pl
from jax.experimental.pallas import tpu as pltpu
from jax.experimental.pallas import tpu_sc as plsc

_BLOCK = 120
_NBUF = 2


@functools.lru_cache(maxsize=None)
def _build(n_nodes: int, n_vocab: int, dim: int):
    info = plsc.get_sparse_core_info()
    nc, ns = info.num_cores, info.num_subcores
    nw = nc * ns

    npb = n_nodes // (nw * _BLOCK)
    nfull = nw * npb
    tail = n_nodes - nfull * _BLOCK
    assert tail % 8 == 0
    tail_sub = 80
    n_tail_workers = tail // tail_sub if tail else 0
    assert n_tail_workers <= nw and (tail == 0 or tail % tail_sub == 0)
    assert npb % _NBUF == 0

    mesh = plsc.VectorSubcoreMesh(core_axis_name="c", subcore_axis_name="s")

    @functools.partial(
        pl.kernel,
        out_type=jax.ShapeDtypeStruct((n_nodes, dim), jnp.float32),
        mesh=mesh,
        scratch_types=[
            pltpu.VMEM_SHARED((n_vocab, dim), jnp.float32),
            pltpu.VMEM((_NBUF, _BLOCK), jnp.int32),
            pltpu.VMEM((_NBUF, _BLOCK, dim), jnp.float32),
            pltpu.SemaphoreType.DMA,
            pltpu.SemaphoreType.DMA,
            pltpu.SemaphoreType.DMA,
        ],
    )
    def emb_kernel(x_hbm, w_hbm, out_hbm, table_s, idx_v, rows_v,
                   sem_g, sem_w0, sem_w1):
        sid = lax.axis_index("s")
        wid = sid * nc + lax.axis_index("c")

        @pl.when(sid == 0)
        def _():
            pltpu.sync_copy(w_hbm, table_s)

        plsc.subcore_barrier()

        w_sems = (sem_w0, sem_w1)

        def out_slot(t):
            return out_hbm.at[pl.ds((wid + t * nw) * _BLOCK, _BLOCK)]

        def fetch_and_gather(t, b):
            pltpu.sync_copy(x_hbm.at[pl.ds((wid + t * nw) * _BLOCK, _BLOCK)],
                            idx_v.at[b])
            pltpu.async_copy(table_s.at[idx_v.at[b]], rows_v.at[b],
                             sem_g).wait()

        def loop_body(p, carry):
            for b in range(_NBUF):
                t = p * _NBUF + b

                @pl.when(p >= 1)
                def _():
                    pltpu.make_async_copy(rows_v.at[b], out_slot(t),
                                          w_sems[b]).wait()

                fetch_and_gather(t, b)
                pltpu.async_copy(rows_v.at[b], out_slot(t), w_sems[b])
            return carry

        lax.fori_loop(0, npb // _NBUF, loop_body, 0)

        for b in range(_NBUF):
            pltpu.make_async_copy(rows_v.at[b], out_slot(npb - _NBUF + b),
                                  w_sems[b]).wait()

        if tail:
            @pl.when(wid >= nw - n_tail_workers)
            def _():
                k = wid - (nw - n_tail_workers)
                base = nfull * _BLOCK + k * tail_sub
                pltpu.sync_copy(x_hbm.at[pl.ds(base, tail_sub)],
                                idx_v.at[0, pl.ds(0, tail_sub)])
                pltpu.async_copy(
                    table_s.at[idx_v.at[0, pl.ds(0, tail_sub)]],
                    rows_v.at[0, pl.ds(0, tail_sub)], sem_g).wait()
                pltpu.sync_copy(rows_v.at[0, pl.ds(0, tail_sub)],
                                out_hbm.at[pl.ds(base, tail_sub)])

    return emb_kernel


def kernel(x, weight):
    n_nodes = x.shape[0]
    n_vocab, dim = weight.shape
    emb = _build(n_nodes, n_vocab, dim)
    return emb(x.astype(jnp.int32), weight)


# speedup vs baseline: 6.6701x; 1.2248x over previous
import functools

import jax
import jax.numpy as jnp
from jax import lax
from jax.experimental import pallas as pl
from jax.experimental.pallas import tpu as pltpu
from jax.experimental.pallas import tpu_sc as plsc

_BLOCK = 120
_NBUF = 2


@functools.lru_cache(maxsize=None)
def _build(n_nodes: int, n_vocab: int, dim: int):
    info = plsc.get_sparse_core_info()
    nc, ns = info.num_cores, info.num_subcores
    nw = nc * ns

    npb = n_nodes // (nw * _BLOCK)
    span = npb * _BLOCK
    tail = n_nodes - nw * span
    assert span % 8 == 0 and tail % 8 == 0
    tail_sub = 80
    n_tail_workers = tail // tail_sub if tail else 0
    assert n_tail_workers <= nw and (tail == 0 or tail % tail_sub == 0)
    assert npb % _NBUF == 0

    mesh = plsc.VectorSubcoreMesh(core_axis_name="c", subcore_axis_name="s")

    @functools.partial(
        pl.kernel,
        out_type=jax.ShapeDtypeStruct((n_nodes, dim), jnp.float32),
        mesh=mesh,
        scratch_types=[
            pltpu.VMEM_SHARED((n_vocab, dim), jnp.float32),
            pltpu.VMEM((npb * _BLOCK,), jnp.int32),
            pltpu.VMEM((_NBUF, _BLOCK, dim), jnp.float32),
            pltpu.VMEM((tail_sub,), jnp.int32),
            pltpu.SemaphoreType.DMA,
            pltpu.SemaphoreType.DMA,
            pltpu.SemaphoreType.DMA,
        ],
    )
    def emb_kernel(x_hbm, w_hbm, out_hbm, table_s, idx_v, rows_v, tidx_v,
                   sem_g, sem_w0, sem_w1):
        sid = lax.axis_index("s")
        wid = sid * nc + lax.axis_index("c")
        base = wid * span

        @pl.when(sid == 0)
        def _():
            pltpu.sync_copy(w_hbm, table_s)

        pltpu.sync_copy(x_hbm.at[pl.ds(base, span)], idx_v)

        plsc.subcore_barrier()

        w_sems = (sem_w0, sem_w1)

        def out_slot(t):
            return out_hbm.at[pl.ds(base + t * _BLOCK, _BLOCK)]

        def loop_body(p, carry):
            for b in range(_NBUF):
                t = p * _NBUF + b

                @pl.when(p >= 1)
                def _():
                    pltpu.make_async_copy(rows_v.at[b], out_slot(t),
                                          w_sems[b]).wait()

                pltpu.async_copy(
                    table_s.at[idx_v.at[pl.ds(t * _BLOCK, _BLOCK)]],
                    rows_v.at[b], sem_g).wait()
                pltpu.async_copy(rows_v.at[b], out_slot(t), w_sems[b])
            return carry

        lax.fori_loop(0, npb // _NBUF, loop_body, 0)

        for b in range(_NBUF):
            pltpu.make_async_copy(rows_v.at[b], out_slot(npb - _NBUF + b),
                                  w_sems[b]).wait()

        if tail:
            @pl.when(wid >= nw - n_tail_workers)
            def _():
                k = wid - (nw - n_tail_workers)
                tbase = nw * span + k * tail_sub
                pltpu.sync_copy(x_hbm.at[pl.ds(tbase, tail_sub)], tidx_v)
                pltpu.async_copy(table_s.at[tidx_v],
                                 rows_v.at[0, pl.ds(0, tail_sub)],
                                 sem_g).wait()
                pltpu.sync_copy(rows_v.at[0, pl.ds(0, tail_sub)],
                                out_hbm.at[pl.ds(tbase, tail_sub)])

    return emb_kernel


def kernel(x, weight):
    n_nodes = x.shape[0]
    n_vocab, dim = weight.shape
    emb = _build(n_nodes, n_vocab, dim)
    return emb(x.astype(jnp.int32), weight)
